# baseline (device time: 107833 ns/iter reference)
import jax
import jax.numpy as jnp
from jax import lax
from jax.experimental import pallas as pl
from jax.experimental.pallas import tpu as pltpu


def kernel(x, pi):
    def body(pi_ref, x_ref, out_ref, send_sem, recv_sem):
        my_x = lax.axis_index("x")
        my_y = lax.axis_index("y")
        my_z = lax.axis_index("z")
        dst_z = pi_ref[my_z]
        rdma = pltpu.make_async_remote_copy(
            src_ref=x_ref,
            dst_ref=out_ref,
            send_sem=send_sem,
            recv_sem=recv_sem,
            device_id=(my_x, my_y, dst_z),
            device_id_type=pl.DeviceIdType.MESH,
        )
        rdma.start()
        rdma.wait()

    return pl.pallas_call(
        body,
        out_shape=jax.ShapeDtypeStruct(x.shape, x.dtype),
        in_specs=[
            pl.BlockSpec(memory_space=pltpu.SMEM),
            pl.BlockSpec(memory_space=pltpu.VMEM),
        ],
        out_specs=pl.BlockSpec(memory_space=pltpu.VMEM),
        scratch_shapes=[
            pltpu.SemaphoreType.DMA,
            pltpu.SemaphoreType.DMA,
        ],
    )(pi, x)


# device time: 99193 ns/iter; 1.0871x vs baseline; 1.0871x over previous
import jax
import jax.numpy as jnp
from jax import lax
from jax.experimental import pallas as pl
from jax.experimental.pallas import tpu as pltpu

N_Z = 4


def kernel(x, pi):
    def body(pi_ref, x_ref, out_ref, send_sem, recv_sem):
        my_x = lax.axis_index("x")
        my_y = lax.axis_index("y")
        my_z = lax.axis_index("z")
        dst_z = pi_ref[my_z]
        src_z = jnp.int32(0)
        for i in range(N_Z):
            src_z = jnp.where(pi_ref[i] == my_z, jnp.int32(i), src_z)

        barrier_sem = pltpu.get_barrier_semaphore()
        pl.semaphore_signal(
            barrier_sem,
            inc=1,
            device_id=(my_x, my_y, src_z),
            device_id_type=pl.DeviceIdType.MESH,
        )
        pl.semaphore_wait(barrier_sem, 1)

        rdma = pltpu.make_async_remote_copy(
            src_ref=x_ref,
            dst_ref=out_ref,
            send_sem=send_sem,
            recv_sem=recv_sem,
            device_id=(my_x, my_y, dst_z),
            device_id_type=pl.DeviceIdType.MESH,
        )
        rdma.start()
        rdma.wait()

    return pl.pallas_call(
        body,
        out_shape=jax.ShapeDtypeStruct(x.shape, x.dtype),
        in_specs=[
            pl.BlockSpec(memory_space=pltpu.SMEM),
            pl.BlockSpec(memory_space=pl.ANY),
        ],
        out_specs=pl.BlockSpec(memory_space=pl.ANY),
        scratch_shapes=[
            pltpu.SemaphoreType.DMA,
            pltpu.SemaphoreType.DMA,
        ],
        compiler_params=pltpu.CompilerParams(collective_id=0),
    )(pi, x)


# device time: 60110 ns/iter; 1.7939x vs baseline; 1.6502x over previous
import jax
import jax.numpy as jnp
from jax import lax
from jax.experimental import pallas as pl
from jax.experimental.pallas import tpu as pltpu

N_Z = 4
HALF = 512
C = 8
ROWS = HALF // C


def kernel(x, pi):
    def body(pi_ref, x_ref, out_ref, z_send, z_recv, x_send, x_recv):
        my_x = lax.axis_index("x")
        my_y = lax.axis_index("y")
        my_z = lax.axis_index("z")
        dst_z = pi_ref[my_z]
        src_z = jnp.int32(0)
        for i in range(N_Z):
            src_z = jnp.where(pi_ref[i] == my_z, jnp.int32(i), src_z)
        px = 1 - my_x
        base = my_x * HALF

        barrier_sem = pltpu.get_barrier_semaphore()
        pl.semaphore_signal(
            barrier_sem, inc=1,
            device_id=(my_x, my_y, src_z),
            device_id_type=pl.DeviceIdType.MESH,
        )
        pl.semaphore_signal(
            barrier_sem, inc=1,
            device_id=(px, my_y, my_z),
            device_id_type=pl.DeviceIdType.MESH,
        )
        pl.semaphore_wait(barrier_sem, 2)

        z_rdmas = []
        for c in range(C):
            r0 = base + c * ROWS
            rd = pltpu.make_async_remote_copy(
                src_ref=x_ref.at[:, pl.ds(r0, ROWS), :],
                dst_ref=out_ref.at[:, pl.ds(r0, ROWS), :],
                send_sem=z_send.at[c],
                recv_sem=z_recv.at[c],
                device_id=(my_x, my_y, dst_z),
                device_id_type=pl.DeviceIdType.MESH,
            )
            rd.start()
            z_rdmas.append(rd)

        x_rdmas = []
        for c in range(C):
            z_rdmas[c].wait_recv()
            r0 = base + c * ROWS
            rd = pltpu.make_async_remote_copy(
                src_ref=out_ref.at[:, pl.ds(r0, ROWS), :],
                dst_ref=out_ref.at[:, pl.ds(r0, ROWS), :],
                send_sem=x_send.at[c],
                recv_sem=x_recv.at[c],
                device_id=(px, my_y, my_z),
                device_id_type=pl.DeviceIdType.MESH,
            )
            rd.start()
            x_rdmas.append(rd)

        for c in range(C):
            x_rdmas[c].wait_recv()
            z_rdmas[c].wait_send()
            x_rdmas[c].wait_send()

    return pl.pallas_call(
        body,
        out_shape=jax.ShapeDtypeStruct(x.shape, x.dtype),
        in_specs=[
            pl.BlockSpec(memory_space=pltpu.SMEM),
            pl.BlockSpec(memory_space=pl.ANY),
        ],
        out_specs=pl.BlockSpec(memory_space=pl.ANY),
        scratch_shapes=[
            pltpu.SemaphoreType.DMA((C,)),
            pltpu.SemaphoreType.DMA((C,)),
            pltpu.SemaphoreType.DMA((C,)),
            pltpu.SemaphoreType.DMA((C,)),
        ],
        compiler_params=pltpu.CompilerParams(collective_id=0),
    )(pi, x)


# device time: 48823 ns/iter; 2.2087x vs baseline; 1.2312x over previous
import jax
import jax.numpy as jnp
from jax import lax
from jax.experimental import pallas as pl
from jax.experimental.pallas import tpu as pltpu

N_Z = 4
QROWS = 256
C = 8
ROWS = QROWS // C


def kernel(x, pi):
    def body(pi_ref, x_ref, out_ref,
             z_send, z_recv, xo_send, xo_recv,
             yo_send, yo_recv, xd_send, xd_recv):
        my_x = lax.axis_index("x")
        my_y = lax.axis_index("y")
        my_z = lax.axis_index("z")
        dst_z = pi_ref[my_z]
        src_z = jnp.int32(0)
        for i in range(N_Z):
            src_z = jnp.where(pi_ref[i] == my_z, jnp.int32(i), src_z)
        px = 1 - my_x
        yp = lax.rem(my_y, 2)
        py = my_y - yp + (1 - yp)
        qbase = (2 * my_x + yp) * QROWS
        ybase = (2 * my_x + (1 - yp)) * QROWS

        barrier_sem = pltpu.get_barrier_semaphore()
        for dev in [(my_x, my_y, src_z), (px, my_y, my_z), (my_x, py, my_z)]:
            pl.semaphore_signal(
                barrier_sem, inc=1,
                device_id=dev,
                device_id_type=pl.DeviceIdType.MESH,
            )
        pl.semaphore_wait(barrier_sem, 3)

        def copy(src_row0, dst_row0, send_sem, recv_sem, dev):
            return pltpu.make_async_remote_copy(
                src_ref=out_ref.at[:, pl.ds(src_row0, ROWS), :],
                dst_ref=out_ref.at[:, pl.ds(dst_row0, ROWS), :],
                send_sem=send_sem,
                recv_sem=recv_sem,
                device_id=dev,
                device_id_type=pl.DeviceIdType.MESH,
            )

        z_rdmas = []
        for c in range(C):
            r0 = qbase + c * ROWS
            rd = pltpu.make_async_remote_copy(
                src_ref=x_ref.at[:, pl.ds(r0, ROWS), :],
                dst_ref=out_ref.at[:, pl.ds(r0, ROWS), :],
                send_sem=z_send.at[c],
                recv_sem=z_recv.at[c],
                device_id=(my_x, my_y, dst_z),
                device_id_type=pl.DeviceIdType.MESH,
            )
            rd.start()
            z_rdmas.append(rd)

        xo_rdmas, yo_rdmas, xd_rdmas = [], [], []
        for c in range(C):
            z_rdmas[c].wait_recv()
            r0 = qbase + c * ROWS
            xo = copy(r0, r0, xo_send.at[c], xo_recv.at[c], (px, my_y, my_z))
            xo.start()
            xo_rdmas.append(xo)
            yo = copy(r0, r0, yo_send.at[c], yo_recv.at[c], (my_x, py, my_z))
            yo.start()
            yo_rdmas.append(yo)
            if c > 0:
                yo_rdmas[c - 1].wait_recv()
                yr0 = ybase + (c - 1) * ROWS
                xd = copy(yr0, yr0, xd_send.at[c - 1], xd_recv.at[c - 1],
                          (px, my_y, my_z))
                xd.start()
                xd_rdmas.append(xd)
        yo_rdmas[C - 1].wait_recv()
        yr0 = ybase + (C - 1) * ROWS
        xd = copy(yr0, yr0, xd_send.at[C - 1], xd_recv.at[C - 1],
                  (px, my_y, my_z))
        xd.start()
        xd_rdmas.append(xd)

        for c in range(C):
            xo_rdmas[c].wait_recv()
            xd_rdmas[c].wait_recv()
            z_rdmas[c].wait_send()
            xo_rdmas[c].wait_send()
            yo_rdmas[c].wait_send()
            xd_rdmas[c].wait_send()

    return pl.pallas_call(
        body,
        out_shape=jax.ShapeDtypeStruct(x.shape, x.dtype),
        in_specs=[
            pl.BlockSpec(memory_space=pltpu.SMEM),
            pl.BlockSpec(memory_space=pl.ANY),
        ],
        out_specs=pl.BlockSpec(memory_space=pl.ANY),
        scratch_shapes=[
            pltpu.SemaphoreType.DMA((C,)),
            pltpu.SemaphoreType.DMA((C,)),
            pltpu.SemaphoreType.DMA((C,)),
            pltpu.SemaphoreType.DMA((C,)),
            pltpu.SemaphoreType.DMA((C,)),
            pltpu.SemaphoreType.DMA((C,)),
            pltpu.SemaphoreType.DMA((C,)),
            pltpu.SemaphoreType.DMA((C,)),
        ],
        compiler_params=pltpu.CompilerParams(collective_id=0),
    )(pi, x)
